# baseline (speedup 1.0000x reference)
import jax
import jax.numpy as jnp
from jax import lax
from jax.experimental import pallas as pl
from jax.experimental.pallas import tpu as pltpu

BATCH = 4096
SEQ = 200
FDIM = 64
NMOD = 16
SP = SEQ // 2
W = 2 * FDIM
BB = 128


def _tc_body(ids_ref, feat_ref, ma_ref, mb_ref, table2_ref, out_ref):
    ids_f = ids_ref[...].astype(jnp.float32)
    a = lax.dot_general(ids_f, ma_ref[...], (((1,), (0,)), ((), ())),
                        preferred_element_type=jnp.float32)
    b = lax.dot_general(ids_f, mb_ref[...], (((1,), (0,)), ((), ())),
                        preferred_element_type=jnp.float32)
    iota = lax.broadcasted_iota(jnp.int32, (1, 1, NMOD), 2)
    ai = a.astype(jnp.int32)
    bi = b.astype(jnp.int32)
    oa = (ai[..., None] == iota).astype(jnp.float32)
    ob = (bi[..., None] == iota).astype(jnp.float32)
    o2 = jnp.concatenate([oa, ob], axis=-1)
    emb = lax.dot_general(
        o2.reshape(BB * SP, 2 * NMOD), table2_ref[...],
        (((1,), (0,)), ((), ())), preferred_element_type=jnp.float32)
    out_ref[...] = feat_ref[...] + emb.reshape(BB, SP, W)


@jax.jit
def _tc_call(f3, ids, ma, mb, table2):
    grid = (BATCH // BB,)
    return pl.pallas_call(
        _tc_body,
        grid=grid,
        in_specs=[
            pl.BlockSpec((BB, SEQ), lambda i: (i, 0)),
            pl.BlockSpec((BB, SP, W), lambda i: (i, 0, 0)),
            pl.BlockSpec((SEQ, SP), lambda i: (0, 0)),
            pl.BlockSpec((SEQ, SP), lambda i: (0, 0)),
            pl.BlockSpec((2 * NMOD, W), lambda i: (0, 0)),
        ],
        out_specs=pl.BlockSpec((BB, SP, W), lambda i: (i, 0, 0)),
        out_shape=jax.ShapeDtypeStruct((BATCH, SP, W), jnp.float32),
        compiler_params=pltpu.CompilerParams(
            dimension_semantics=("arbitrary",)),
    )(ids, f3, ma, mb, table2)


def kernel(features, modality_ids, sinusoidal_embedding):
    ids = modality_ids.astype(jnp.int32)
    f3 = features.reshape(BATCH, SP, W)
    eye = jnp.eye(SP, dtype=jnp.float32)
    ma = jnp.zeros((SEQ, SP), jnp.float32).at[0::2, :].set(eye)
    mb = jnp.zeros((SEQ, SP), jnp.float32).at[1::2, :].set(eye)
    z = jnp.zeros((NMOD, FDIM), jnp.float32)
    table2 = jnp.concatenate([
        jnp.concatenate([sinusoidal_embedding, z], axis=1),
        jnp.concatenate([z, sinusoidal_embedding], axis=1),
    ], axis=0)
    out3 = _tc_call(f3, ids, ma, mb, table2)
    return out3.reshape(BATCH, SEQ, FDIM)


# speedup vs baseline: 1.1478x; 1.1478x over previous
import jax
import jax.numpy as jnp
from jax import lax
from jax.experimental import pallas as pl
from jax.experimental.pallas import tpu as pltpu

BATCH = 4096
SEQ = 200
FDIM = 64
NMOD = 16
WIDE = SEQ * FDIM
NG = WIDE // 512
BB = 128


def _tc_body(ids_ref, feat_ref, rep_ref, g_ref, out_ref):
    ids_f = ids_ref[...].astype(jnp.float32)
    rep = lax.dot_general(ids_f, rep_ref[...], (((1,), (0,)), ((), ())),
                          preferred_element_type=jnp.float32)
    repi = rep.astype(jnp.int32)
    li = jnp.bitwise_and(
        lax.broadcasted_iota(jnp.int32, (1, SEQ * NMOD), 1), NMOD - 1)
    oh = (repi == li).astype(jnp.float32)
    g = g_ref[...]
    for grp in range(NG):
        og = oh[:, 128 * grp:128 * (grp + 1)]
        emb = lax.dot_general(og, g, (((1,), (0,)), ((), ())),
                              preferred_element_type=jnp.float32)
        sl = pl.ds(512 * grp, 512)
        out_ref[:, sl] = feat_ref[:, sl] + emb


@jax.jit
def _tc_call(f2, ids, rep_m, g_m):
    grid = (BATCH // BB,)
    return pl.pallas_call(
        _tc_body,
        grid=grid,
        in_specs=[
            pl.BlockSpec((BB, SEQ), lambda i: (i, 0)),
            pl.BlockSpec((BB, WIDE), lambda i: (i, 0)),
            pl.BlockSpec((SEQ, SEQ * NMOD), lambda i: (0, 0)),
            pl.BlockSpec((128, 512), lambda i: (0, 0)),
        ],
        out_specs=pl.BlockSpec((BB, WIDE), lambda i: (i, 0)),
        out_shape=jax.ShapeDtypeStruct((BATCH, WIDE), jnp.float32),
        compiler_params=pltpu.CompilerParams(
            dimension_semantics=("arbitrary",)),
    )(ids, f2, rep_m, g_m)


def kernel(features, modality_ids, sinusoidal_embedding):
    ids = modality_ids.astype(jnp.int32)
    f2 = features.reshape(BATCH, WIDE)
    rep_m = jnp.kron(jnp.eye(SEQ, dtype=jnp.float32),
                     jnp.ones((1, NMOD), jnp.float32))
    z = jnp.zeros((NMOD, FDIM), jnp.float32)
    table2 = jnp.concatenate([
        jnp.concatenate([sinusoidal_embedding, z], axis=1),
        jnp.concatenate([z, sinusoidal_embedding], axis=1),
    ], axis=0)
    g_m = jnp.kron(jnp.eye(4, dtype=jnp.float32), table2)
    out2 = _tc_call(f2, ids, rep_m, g_m)
    return out2.reshape(BATCH, SEQ, FDIM)
